# baseline (speedup 1.0000x reference)
import functools

import jax
import jax.numpy as jnp
from jax import lax
from jax.experimental import pallas as pl
from jax.experimental.pallas import tpu as pltpu
from jax.experimental.pallas import tpu_sc as plsc

NUM_ROWS = 16384
SEQ = 200
DIM = 16

NC = 2
NS = 16
NW = NC * NS

ISLAB = NUM_ROWS // NW
N_IT = ISLAB // 128
JB = 8
N_JB = SEQ // JB
N_OUT_ROWS = SEQ * 2 * (NUM_ROWS // 128) * 8


_GATHER_DNUMS = lax.GatherDimensionNumbers(
    offset_dims=(), collapsed_slice_dims=(0,), start_index_map=(0,)
)


def _permute(x, idx16):
    return lax.gather(
        x,
        idx16[:, None],
        _GATHER_DNUMS,
        slice_sizes=(1,),
        mode=lax.GatherScatterMode.PROMISE_IN_BOUNDS,
    )


def _sc_body(idxt_hbm, tb_hbm, out_hbm, idx_v, tb_v, out_v, sem, sem_idx):
    wid = lax.axis_index("s") * NC + lax.axis_index("c")
    i0 = wid * ISLAB
    pltpu.sync_copy(tb_hbm, tb_v)
    tcol = [tb_v[d] for d in range(DIM)]

    def idx_block(jb):
        return idxt_hbm.at[pl.ds(jb * JB, JB), pl.ds(i0, ISLAB)]

    pltpu.async_copy(idx_block(0), idx_v.at[0], sem_idx)

    def jb_body(jb, _):
        ibuf = jb & 1
        pltpu.make_async_copy(idx_block(jb), idx_v.at[ibuf], sem_idx).wait()

        @pl.when(jb + 1 < N_JB)
        def _():
            pltpu.async_copy(idx_block(jb + 1), idx_v.at[1 - ibuf], sem_idx)

        def j_body(jj, _):
            j = jb * JB + jj
            buf = (jb * JB + jj) % 3

            @pl.when(j >= 3)
            def _():
                for dt in range(2):
                    pltpu.make_async_copy(
                        out_v.at[buf, dt], out_hbm.at[pl.ds(0, 32)], sem
                    ).wait()

            for itl in range(N_IT):
                for sub in range(8):
                    v = idx_v[ibuf, jj, pl.ds(itl * 128 + sub * 16, 16)]
                    for dt in range(2):
                        for ds in range(8):
                            out_v[buf, dt, itl * 8 + ds, pl.ds(sub * 16, 16)] = (
                                _permute(tcol[dt * 8 + ds], v)
                            )
            for dt in range(2):
                n0 = (j * 2 + dt) * 1024 + 32 * wid
                pltpu.async_copy(
                    out_v.at[buf, dt],
                    out_hbm.at[pl.ds(pl.multiple_of(n0, 32), 32)],
                    sem,
                )
            return ()

        lax.fori_loop(0, JB, j_body, ())
        return ()

    lax.fori_loop(0, N_JB, jb_body, ())
    for _ in range(6):
        pltpu.make_async_copy(
            out_v.at[0, 0], out_hbm.at[pl.ds(0, 32)], sem
        ).wait()


@jax.jit
def _sc_embed(idxt, tb):
    mesh = plsc.VectorSubcoreMesh(core_axis_name="c", subcore_axis_name="s")
    f = functools.partial(
        pl.kernel,
        mesh=mesh,
        out_type=jax.ShapeDtypeStruct((N_OUT_ROWS, 128), jnp.float32),
        scratch_types=[
            pltpu.VMEM((2, JB, ISLAB), jnp.int32),
            pltpu.VMEM((DIM, 16), jnp.float32),
            pltpu.VMEM((3, 2, 32, 128), jnp.float32),
            pltpu.SemaphoreType.DMA,
            pltpu.SemaphoreType.DMA,
        ],
    )(_sc_body)
    return f(idxt, tb)


def kernel(action_indices, embedding_table):
    idxt = action_indices.astype(jnp.int32).T
    tb = jnp.concatenate(
        [embedding_table.T, jnp.zeros((DIM, 16 - 4), jnp.float32)], axis=1
    )
    out = _sc_embed(idxt, tb)
    o5 = out.reshape(SEQ, 2, NUM_ROWS // 128, 8, 128)
    return o5.transpose(2, 4, 0, 1, 3).reshape(NUM_ROWS, SEQ, DIM)


# speedup vs baseline: 1.0037x; 1.0037x over previous
import functools

import jax
import jax.numpy as jnp
from jax import lax
from jax.experimental import pallas as pl
from jax.experimental.pallas import tpu as pltpu
from jax.experimental.pallas import tpu_sc as plsc

NUM_ROWS = 16384
SEQ = 200
DIM = 16

NC = 2
NS = 16
NW = NC * NS

ISLAB = NUM_ROWS // NW
N_IT = ISLAB // 128
JB = 8
N_JB = SEQ // JB
N_OUT_ROWS = SEQ * 2 * (NUM_ROWS // 128) * 8


_GATHER_DNUMS = lax.GatherDimensionNumbers(
    offset_dims=(), collapsed_slice_dims=(0,), start_index_map=(0,)
)


def _permute(x, idx16):
    return lax.gather(
        x,
        idx16[:, None],
        _GATHER_DNUMS,
        slice_sizes=(1,),
        mode=lax.GatherScatterMode.PROMISE_IN_BOUNDS,
    )


def _sc_body(idxt_hbm, tb_hbm, out_hbm, idx_v, tb_v, out_v, sem, sem_idx):
    wid = lax.axis_index("s") * NC + lax.axis_index("c")
    i0 = wid * ISLAB
    pltpu.sync_copy(tb_hbm, tb_v)
    tcol = [tb_v[d] for d in range(DIM)]

    def idx_block(jb):
        return idxt_hbm.at[pl.ds(jb * JB, JB), pl.ds(i0, ISLAB)]

    pltpu.async_copy(idx_block(0), idx_v.at[0], sem_idx)

    def jb_body(jb, _):
        ibuf = jb & 1
        pltpu.make_async_copy(idx_block(jb), idx_v.at[ibuf], sem_idx).wait()

        @pl.when(jb + 1 < N_JB)
        def _():
            pltpu.async_copy(idx_block(jb + 1), idx_v.at[1 - ibuf], sem_idx)

        def j_body(jj, _):
            j = jb * JB + jj
            buf = jj & 1

            @pl.when(j >= 2)
            def _():
                for dt in range(2):
                    pltpu.make_async_copy(
                        out_v.at[buf, dt], out_hbm.at[pl.ds(0, 32)], sem
                    ).wait()

            for itl in range(N_IT):
                for sub in range(8):
                    v = idx_v[ibuf, jj, pl.ds(itl * 128 + sub * 16, 16)]
                    for dt in range(2):
                        for ds in range(8):
                            out_v[buf, dt, itl * 8 + ds, pl.ds(sub * 16, 16)] = (
                                _permute(tcol[dt * 8 + ds], v)
                            )
            for dt in range(2):
                n0 = (j * 2 + dt) * 1024 + 32 * wid
                pltpu.async_copy(
                    out_v.at[buf, dt],
                    out_hbm.at[pl.ds(pl.multiple_of(n0, 32), 32)],
                    sem,
                )
            return ()

        lax.fori_loop(0, JB, j_body, ())
        return ()

    lax.fori_loop(0, N_JB, jb_body, ())
    for _ in range(4):
        pltpu.make_async_copy(
            out_v.at[0, 0], out_hbm.at[pl.ds(0, 32)], sem
        ).wait()


@jax.jit
def _sc_embed(idxt, tb):
    mesh = plsc.VectorSubcoreMesh(core_axis_name="c", subcore_axis_name="s")
    f = functools.partial(
        pl.kernel,
        mesh=mesh,
        out_type=jax.ShapeDtypeStruct((N_OUT_ROWS, 128), jnp.float32),
        scratch_types=[
            pltpu.VMEM((2, JB, ISLAB), jnp.int32),
            pltpu.VMEM((DIM, 16), jnp.float32),
            pltpu.VMEM((2, 2, 32, 128), jnp.float32),
            pltpu.SemaphoreType.DMA,
            pltpu.SemaphoreType.DMA,
        ],
    )(_sc_body)
    return f(idxt, tb)


def kernel(action_indices, embedding_table):
    idxt = action_indices.astype(jnp.int32).T
    tb = jnp.concatenate(
        [embedding_table.T, jnp.zeros((DIM, 16 - 4), jnp.float32)], axis=1
    )
    out = _sc_embed(idxt, tb)
    o5 = out.reshape(SEQ, 2, NUM_ROWS // 128, 8, 128)
    return o5.transpose(2, 4, 0, 1, 3).reshape(NUM_ROWS, SEQ, DIM)
